# baseline (device time: 3559 ns/iter reference)
import jax
import jax.numpy as jnp
from jax import lax
from jax.experimental import pallas as pl
from jax.experimental.pallas import tpu as pltpu

EPS = 1e-5


def kernel(x, gamma, beta):
    m, n_loc = x.shape
    n_global = n_loc * 4

    def body(x_ref, g_ref, b_ref, out_ref):
        x = x_ref[:, :]
        s = jnp.sum(x, axis=1, keepdims=True) * 4.0
        sq = jnp.sum(x * x, axis=1, keepdims=True) * 4.0
        mean = s / n_global
        var = sq / n_global - mean * mean
        inv = lax.rsqrt(var + EPS)
        g = g_ref[:].reshape(1, n_loc)
        b = b_ref[:].reshape(1, n_loc)
        out_ref[:, :] = g * ((x - mean) * inv) + b

    return pl.pallas_call(
        body,
        out_shape=jax.ShapeDtypeStruct((m, n_loc), jnp.float32),
        in_specs=[
            pl.BlockSpec(memory_space=pltpu.VMEM),
            pl.BlockSpec(memory_space=pltpu.VMEM),
            pl.BlockSpec(memory_space=pltpu.VMEM),
        ],
        out_specs=pl.BlockSpec(memory_space=pltpu.VMEM),
    )(x, gamma, beta)
